# baseline (speedup 1.0000x reference)
import functools

import jax
import jax.numpy as jnp
from jax import lax
from jax.experimental import pallas as pl
from jax.experimental.pallas import tpu as pltpu
from jax.experimental.pallas import tpu_sc as plsc

_H = 26
_D = 64
_LANES = 16


def _body(chunk, group, n_groups, nbuf,
          ids_hbm, offs_hbm, table_hbm, out_hbm,
          idx_v, offs_v, rows_v, gsem, wsem):
  wid = lax.axis_index("s") * 2 + lax.axis_index("c")
  base = wid * chunk

  pltpu.sync_copy(ids_hbm.at[pl.ds(base, chunk)], idx_v)
  pltpu.sync_copy(offs_hbm, offs_v)

  def shift(j, _):
    p = j * _LANES
    off = offs_v[pl.ds(lax.rem(p, 8 * _H), _LANES)]
    idx_v[pl.ds(p, _LANES)] = idx_v[pl.ds(p, _LANES)] + off
    return 0

  lax.fori_loop(0, chunk // _LANES, shift, 0, unroll=4)

  def gather_desc(g, b):
    return pltpu.make_async_copy(
        table_hbm.at[idx_v.at[pl.ds(g * group, group)]], rows_v.at[b],
        gsem.at[b])

  def write_desc(g, b):
    return pltpu.make_async_copy(
        rows_v.at[b], out_hbm.at[pl.ds(base + g * group, group)], wsem.at[b])

  for b in range(nbuf):
    gather_desc(b, b).start()

  def step(s, _):
    g0 = s * nbuf
    for b in range(nbuf):
      gather_desc(g0 + b, b).wait()
      write_desc(g0 + b, b).start()
    for b in range(nbuf):
      write_desc(g0 + b, b).wait()
      gather_desc(g0 + nbuf + b, b).start()
    return 0

  n_super = n_groups // nbuf
  lax.fori_loop(0, n_super - 1, step, 0)

  g0 = (n_super - 1) * nbuf
  for b in range(nbuf):
    gather_desc(g0 + b, b).wait()
    write_desc(g0 + b, b).start()
  for b in range(nbuf):
    write_desc(g0 + b, b).wait()


@jax.jit
def _mhe(hash_ids, table, offsets):
  bh = hash_ids.shape[0] * hash_ids.shape[1]
  info = plsc.get_sparse_core_info()
  nw = info.num_cores * info.num_subcores
  chunk = bh // nw
  group = 128
  n_groups = chunk // group
  nbuf = 4

  ids_flat = hash_ids.reshape(bh)
  offs_tiled = jnp.tile(offsets, 8)

  mesh = plsc.VectorSubcoreMesh(core_axis_name="c", subcore_axis_name="s")
  body = functools.partial(_body, chunk, group, n_groups, nbuf)
  out = pl.kernel(
      body,
      out_type=jax.ShapeDtypeStruct((bh, _D), jnp.float32),
      mesh=mesh,
      compiler_params=pltpu.CompilerParams(use_tc_tiling_on_sc=False),
      scratch_types=[
          pltpu.VMEM((chunk,), jnp.int32),
          pltpu.VMEM((8 * _H,), jnp.int32),
          pltpu.VMEM((nbuf, group, _D), jnp.float32),
          pltpu.SemaphoreType.DMA((nbuf,)),
          pltpu.SemaphoreType.DMA((nbuf,)),
      ],
  )(ids_flat, offs_tiled, table)
  return out.reshape(hash_ids.shape[0], hash_ids.shape[1], _D)


def kernel(hash_ids, table, offsets):
  return _mhe(hash_ids, table, offsets)
